# baseline (speedup 1.0000x reference)
import functools

import jax
import jax.numpy as jnp
from jax import lax
from jax.experimental import pallas as pl
from jax.experimental.pallas import tpu as pltpu
from jax.experimental.pallas import tpu_sc as plsc

LAMBDA_ORTHO = 0.0001
M = 1.0
T = 16
KP = 112
_INF = float("inf")


def _dense_block(v_ref, vh_ref, g_ref, f_ref, neg_ref,
                 dist_ref, gp_ref, base_ref, rest_ref):
    v = v_ref[...]
    vh = vh_ref[...]
    F = f_ref[...]
    neg = neg_ref[...]

    BR = v.shape[0]
    K = F.shape[0]
    N = neg.shape[0]

    base = jnp.sqrt(jnp.sum((vh - v) ** 2, axis=1, keepdims=True))
    vh_sq = jnp.sum(vh * vh, axis=1, keepdims=True)

    neg_sq = jnp.sum(neg * neg, axis=1)[None, :]
    dot_n = lax.dot_general(vh, neg, (((1,), (1,)), ((), ())),
                            preferred_element_type=jnp.float32)
    neg_dist = jnp.sqrt(jnp.maximum(vh_sq + neg_sq - 2.0 * dot_n, 0.0))
    ju = jnp.sum(jnp.maximum(1.0 + base - neg_dist, 0.0), axis=1) / N

    f_sq = jnp.sum(F * F, axis=1)[None, :]
    dot_f = lax.dot_general(vh, F, (((1,), (1,)), ((), ())),
                            preferred_element_type=jnp.float32)
    dist_f = jnp.sqrt(jnp.maximum(vh_sq + f_sq - 2.0 * dot_f, 0.0))

    gram = lax.dot_general(F, F, (((1,), (1,)), ((), ())),
                           preferred_element_type=jnp.float32)
    r = lax.broadcasted_iota(jnp.int32, (K, K), 0)
    c = lax.broadcasted_iota(jnp.int32, (K, K), 1)
    eye = jnp.where(r == c, 1.0, 0.0).astype(jnp.float32)
    ortho = jnp.sum(jnp.abs(gram - eye))

    dist_ref[...] = jnp.concatenate(
        [dist_f, jnp.zeros((BR, KP - K), jnp.float32)], axis=1)
    gp_ref[...] = jnp.concatenate(
        [g_ref[...], jnp.full((BR, KP - K), _INF, jnp.float32)], axis=1)
    base_ref[...] = base[:, 0]
    rest_ref[...] = ju + LAMBDA_ORTHO * ortho * ortho


def _dense_stage(v, vhat, g, F, negatives, block_rows):
    B, D = v.shape
    K = F.shape[0]
    N = negatives.shape[0]
    grid = (B // block_rows,)
    return pl.pallas_call(
        _dense_block,
        grid=grid,
        in_specs=[
            pl.BlockSpec((block_rows, D), lambda i: (i, 0)),
            pl.BlockSpec((block_rows, D), lambda i: (i, 0)),
            pl.BlockSpec((block_rows, K), lambda i: (i, 0)),
            pl.BlockSpec((K, D), lambda i: (0, 0)),
            pl.BlockSpec((N, D), lambda i: (0, 0)),
        ],
        out_specs=[
            pl.BlockSpec((block_rows, KP), lambda i: (i, 0)),
            pl.BlockSpec((block_rows, KP), lambda i: (i, 0)),
            pl.BlockSpec((block_rows,), lambda i: (i,)),
            pl.BlockSpec((block_rows,), lambda i: (i,)),
        ],
        out_shape=[
            jax.ShapeDtypeStruct((B, KP), jnp.float32),
            jax.ShapeDtypeStruct((B, KP), jnp.float32),
            jax.ShapeDtypeStruct((B,), jnp.float32),
            jax.ShapeDtypeStruct((B,), jnp.float32),
        ],
    )(v, vhat, g, F, negatives)


def _bottom16_row(g_v, d_v, row):
    nchunk = KP // 16
    chunks = []
    for cki in range(nchunk):
        k = g_v[row, pl.ds(cki * 16, 16)]
        v = d_v[row, pl.ds(cki * 16, 16)]
        chunks.append(plsc.sort_key_val(k, v))

    def merge(a, b):
        ak, av = a
        bk, bv = b
        rk = lax.rev(bk, (0,))
        rv = lax.rev(bv, (0,))
        take_a = ak <= rk
        mk = jnp.where(take_a, ak, rk)
        mv = jnp.where(take_a, av, rv)
        return plsc.sort_key_val(mk, mv)

    while len(chunks) > 1:
        nxt = [merge(chunks[i], chunks[i + 1])
               for i in range(0, len(chunks) - 1, 2)]
        if len(chunks) % 2:
            nxt.append(chunks[-1])
        chunks = nxt
    return chunks[0]


def _make_sc_stage(B):
    info = plsc.get_sparse_core_info()
    NC, NS = info.num_cores, info.num_subcores
    NW = NC * NS
    RW = B // NW
    CR = 128
    NCH = RW // CR
    GROUPS = CR // 16

    mesh = plsc.VectorSubcoreMesh(core_axis_name="c", subcore_axis_name="s")

    @functools.partial(
        pl.kernel,
        out_type=jax.ShapeDtypeStruct((B,), jnp.float32),
        mesh=mesh,
        compiler_params=pltpu.CompilerParams(needs_layout_passes=False),
        scratch_types=[
            pltpu.VMEM((2, CR, KP), jnp.float32),
            pltpu.VMEM((2, CR, KP), jnp.float32),
            pltpu.VMEM((RW,), jnp.float32),
            pltpu.VMEM((RW,), jnp.float32),
            pltpu.VMEM((RW,), jnp.float32),
            pltpu.SemaphoreType.DMA,
            pltpu.SemaphoreType.DMA,
        ],
    )
    def sc_topk(g_hbm, dist_hbm, base_hbm, rest_hbm, out_hbm,
                g_v, d_v, b_v, r_v, o_v, sem0, sem1):
        wid = lax.axis_index("s") * NC + lax.axis_index("c")
        row0 = wid * RW
        sems = (sem0, sem1)

        def start(ci, slot):
            rows = pl.ds(row0 + ci * CR, CR)
            dg = pltpu.async_copy(g_hbm.at[rows], g_v.at[slot], sems[slot])
            dd = pltpu.async_copy(dist_hbm.at[rows], d_v.at[slot], sems[slot])
            return dg, dd

        pltpu.sync_copy(base_hbm.at[pl.ds(row0, RW)], b_v)
        pltpu.sync_copy(rest_hbm.at[pl.ds(row0, RW)], r_v)

        lane = lax.iota(jnp.int32, 16)
        pending = start(0, 0)

        for ci in range(NCH):
            slot = ci % 2
            if ci + 1 < NCH:
                nxt = start(ci + 1, 1 - slot)
            for dsc in pending:
                dsc.wait()
            if ci + 1 < NCH:
                pending = nxt

            def group_body(gi, carry, ci=ci, slot=slot):
                acc = jnp.zeros((16,), jnp.float32)
                for j in range(16):
                    row = gi * 16 + j
                    bk, bv = _bottom16_row(g_v.at[slot], d_v.at[slot], row)
                    s = jnp.sum(bk)
                    g_t = bk / (s + 1e-10)
                    one_m = 1.0 - g_t
                    m_t = M * one_m * one_m
                    arow = jnp.full((16,), ci * CR + row, jnp.int32)
                    basev = plsc.load_gather(b_v, [arow])
                    restv = plsc.load_gather(r_v, [arow])
                    hinge = jnp.maximum(m_t + basev - bv, 0.0)
                    jt = jnp.sum(hinge) * (1.0 / T)
                    acc = jnp.where(lane == j, jt + restv, acc)
                o_v[pl.ds(ci * CR + gi * 16, 16)] = acc
                return carry

            lax.fori_loop(0, GROUPS, group_body, 0)
        pltpu.sync_copy(o_v, out_hbm.at[pl.ds(row0, RW)])

    return sc_topk


@functools.partial(jax.jit, static_argnames=("block_rows", "stages"))
def _run(v, vhat, g, F, negatives, block_rows=1024, stages=2):
    B = v.shape[0]
    SB = B // stages
    sc = _make_sc_stage(SB)
    outs = []
    for i in range(stages):
        sl = slice(i * SB, (i + 1) * SB)
        dist, g_pad, base, rest = _dense_stage(
            v[sl], vhat[sl], g[sl], F, negatives, block_rows)
        outs.append(sc(g_pad, dist, base, rest))
    return jnp.concatenate(outs)


def kernel(v, vhat, d, g, F, negatives):
    del d
    return _run(v, vhat, g, F, negatives)


# speedup vs baseline: 1.1263x; 1.1263x over previous
import functools

import jax
import jax.numpy as jnp
from jax import lax
from jax.experimental import pallas as pl
from jax.experimental.pallas import tpu as pltpu
from jax.experimental.pallas import tpu_sc as plsc

LAMBDA_ORTHO = 0.0001
M = 1.0
T = 16
K = 100
_INF = float("inf")


def _dense_block(v_ref, vh_ref, f_ref, neg_ref,
                 dist_ref, base_ref, rest_ref):
    v = v_ref[...]
    vh = vh_ref[...]
    F = f_ref[...]
    neg = neg_ref[...]
    N = neg.shape[0]

    base = jnp.sqrt(jnp.sum((vh - v) ** 2, axis=1, keepdims=True))
    vh_sq = jnp.sum(vh * vh, axis=1, keepdims=True)

    neg_sq = jnp.sum(neg * neg, axis=1)[None, :]
    dot_n = lax.dot_general(vh, neg, (((1,), (1,)), ((), ())),
                            preferred_element_type=jnp.float32)
    neg_dist = jnp.sqrt(jnp.maximum(vh_sq + neg_sq - 2.0 * dot_n, 0.0))
    ju = jnp.sum(jnp.maximum(1.0 + base - neg_dist, 0.0), axis=1) / N

    f_sq = jnp.sum(F * F, axis=1)[None, :]
    dot_f = lax.dot_general(vh, F, (((1,), (1,)), ((), ())),
                            preferred_element_type=jnp.float32)
    dist_f = jnp.sqrt(jnp.maximum(vh_sq + f_sq - 2.0 * dot_f, 0.0))

    gram = lax.dot_general(F, F, (((1,), (1,)), ((), ())),
                           preferred_element_type=jnp.float32)
    r = lax.broadcasted_iota(jnp.int32, (K, K), 0)
    c = lax.broadcasted_iota(jnp.int32, (K, K), 1)
    eye = jnp.where(r == c, 1.0, 0.0).astype(jnp.float32)
    ortho = jnp.sum(jnp.abs(gram - eye))

    dist_ref[...] = dist_f
    base_ref[...] = base[:, 0]
    rest_ref[...] = ju + LAMBDA_ORTHO * ortho * ortho


def _dense_stage(v, vhat, F, negatives, block_rows):
    B, D = v.shape
    N = negatives.shape[0]
    grid = (B // block_rows,)
    return pl.pallas_call(
        _dense_block,
        grid=grid,
        in_specs=[
            pl.BlockSpec((block_rows, D), lambda i: (i, 0)),
            pl.BlockSpec((block_rows, D), lambda i: (i, 0)),
            pl.BlockSpec((K, D), lambda i: (0, 0)),
            pl.BlockSpec((N, D), lambda i: (0, 0)),
        ],
        out_specs=[
            pl.BlockSpec((block_rows, K), lambda i: (i, 0)),
            pl.BlockSpec((block_rows,), lambda i: (i,)),
            pl.BlockSpec((block_rows,), lambda i: (i,)),
        ],
        out_shape=[
            jax.ShapeDtypeStruct((B, K), jnp.float32),
            jax.ShapeDtypeStruct((B,), jnp.float32),
            jax.ShapeDtypeStruct((B,), jnp.float32),
        ],
    )(v, vhat, F, negatives)


_CHUNK_OFFS = (0, 16, 32, 48, 64, 80, 84)


def _bottom16_row(g_v, d_v, row, lane):
    chunks = []
    for cki, off in enumerate(_CHUNK_OFFS):
        k = g_v[row, pl.ds(off, 16)]
        v = d_v[row, pl.ds(off, 16)]
        if off % 16:
            k = jnp.where(lane < (16 - K % 16), _INF, k)
        chunks.append(plsc.sort_key_val(k, v))

    def merge(a, b):
        ak, av = a
        bk, bv = b
        rk = lax.rev(bk, (0,))
        rv = lax.rev(bv, (0,))
        take_a = ak <= rk
        mk = jnp.where(take_a, ak, rk)
        mv = jnp.where(take_a, av, rv)
        return plsc.sort_key_val(mk, mv)

    while len(chunks) > 1:
        nxt = [merge(chunks[i], chunks[i + 1])
               for i in range(0, len(chunks) - 1, 2)]
        if len(chunks) % 2:
            nxt.append(chunks[-1])
        chunks = nxt
    return chunks[0]


def _make_sc_stage(B):
    info = plsc.get_sparse_core_info()
    NC, NS = info.num_cores, info.num_subcores
    NW = NC * NS
    RW = B // NW
    CR = 128
    NCH = RW // CR
    GROUPS = CR // 16

    mesh = plsc.VectorSubcoreMesh(core_axis_name="c", subcore_axis_name="s")

    @functools.partial(
        pl.kernel,
        out_type=jax.ShapeDtypeStruct((B,), jnp.float32),
        mesh=mesh,
        compiler_params=pltpu.CompilerParams(needs_layout_passes=False),
        scratch_types=[
            pltpu.VMEM((2, CR, K), jnp.float32),
            pltpu.VMEM((2, CR, K), jnp.float32),
            pltpu.VMEM((RW,), jnp.float32),
            pltpu.VMEM((RW,), jnp.float32),
            pltpu.VMEM((RW,), jnp.float32),
            pltpu.SemaphoreType.DMA,
            pltpu.SemaphoreType.DMA,
        ],
    )
    def sc_topk(g_hbm, dist_hbm, base_hbm, rest_hbm, out_hbm,
                g_v, d_v, b_v, r_v, o_v, sem0, sem1):
        wid = lax.axis_index("s") * NC + lax.axis_index("c")
        row0 = wid * RW
        sems = (sem0, sem1)

        def start(ci, slot):
            rows = pl.ds(row0 + ci * CR, CR)
            dg = pltpu.async_copy(g_hbm.at[rows], g_v.at[slot], sems[slot])
            dd = pltpu.async_copy(dist_hbm.at[rows], d_v.at[slot], sems[slot])
            return dg, dd

        pltpu.sync_copy(base_hbm.at[pl.ds(row0, RW)], b_v)
        pltpu.sync_copy(rest_hbm.at[pl.ds(row0, RW)], r_v)

        lane = lax.iota(jnp.int32, 16)
        pending = start(0, 0)

        for ci in range(NCH):
            slot = ci % 2
            if ci + 1 < NCH:
                nxt = start(ci + 1, 1 - slot)
            for dsc in pending:
                dsc.wait()
            if ci + 1 < NCH:
                pending = nxt

            def group_body(gi, carry, ci=ci, slot=slot):
                acc = jnp.zeros((16,), jnp.float32)
                for j in range(16):
                    row = gi * 16 + j
                    bk, bv = _bottom16_row(g_v.at[slot], d_v.at[slot],
                                           row, lane)
                    s = jnp.sum(bk)
                    g_t = bk / (s + 1e-10)
                    one_m = 1.0 - g_t
                    m_t = M * one_m * one_m
                    arow = jnp.full((16,), ci * CR + row, jnp.int32)
                    basev = plsc.load_gather(b_v, [arow])
                    restv = plsc.load_gather(r_v, [arow])
                    hinge = jnp.maximum(m_t + basev - bv, 0.0)
                    jt = jnp.sum(hinge) * (1.0 / T)
                    acc = jnp.where(lane == j, jt + restv, acc)
                o_v[pl.ds(ci * CR + gi * 16, 16)] = acc
                return carry

            lax.fori_loop(0, GROUPS, group_body, 0)
        pltpu.sync_copy(o_v, out_hbm.at[pl.ds(row0, RW)])

    return sc_topk


@functools.partial(jax.jit, static_argnames=("block_rows",))
def _run(v, vhat, g, F, negatives, block_rows=1024):
    B = v.shape[0]
    dist, base, rest = _dense_stage(v, vhat, F, negatives, block_rows)
    return _make_sc_stage(B)(g, dist, base, rest)


def kernel(v, vhat, d, g, F, negatives):
    del d
    return _run(v, vhat, g, F, negatives)


# speedup vs baseline: 1.2756x; 1.1325x over previous
import functools

import jax
import jax.numpy as jnp
from jax import lax
from jax.experimental import pallas as pl
from jax.experimental.pallas import tpu as pltpu
from jax.experimental.pallas import tpu_sc as plsc

LAMBDA_ORTHO = 0.0001
M = 1.0
T = 16
K = 100
_INF = float("inf")


def _dense_block(v_ref, vh_ref, f_ref, neg_ref,
                 dist_ref, rest_ref):
    v = v_ref[...]
    vh = vh_ref[...]
    F = f_ref[...]
    neg = neg_ref[...]
    N = neg.shape[0]

    base = jnp.sqrt(jnp.sum((vh - v) ** 2, axis=1, keepdims=True))
    vh_sq = jnp.sum(vh * vh, axis=1, keepdims=True)

    neg_sq = jnp.sum(neg * neg, axis=1)[None, :]
    dot_n = lax.dot_general(vh, neg, (((1,), (1,)), ((), ())),
                            preferred_element_type=jnp.float32)
    neg_dist = jnp.sqrt(jnp.maximum(vh_sq + neg_sq - 2.0 * dot_n, 0.0))
    ju = jnp.sum(jnp.maximum(1.0 + base - neg_dist, 0.0), axis=1) / N

    f_sq = jnp.sum(F * F, axis=1)[None, :]
    dot_f = lax.dot_general(vh, F, (((1,), (1,)), ((), ())),
                            preferred_element_type=jnp.float32)
    dist_f = jnp.sqrt(jnp.maximum(vh_sq + f_sq - 2.0 * dot_f, 0.0))

    gram = lax.dot_general(F, F, (((1,), (1,)), ((), ())),
                           preferred_element_type=jnp.float32)
    r = lax.broadcasted_iota(jnp.int32, (K, K), 0)
    c = lax.broadcasted_iota(jnp.int32, (K, K), 1)
    eye = jnp.where(r == c, 1.0, 0.0).astype(jnp.float32)
    ortho = jnp.sum(jnp.abs(gram - eye))

    dist_ref[...] = dist_f - base
    rest_ref[...] = ju + LAMBDA_ORTHO * ortho * ortho


def _dense_stage(v, vhat, F, negatives, block_rows):
    B, D = v.shape
    N = negatives.shape[0]
    grid = (B // block_rows,)
    return pl.pallas_call(
        _dense_block,
        grid=grid,
        in_specs=[
            pl.BlockSpec((block_rows, D), lambda i: (i, 0)),
            pl.BlockSpec((block_rows, D), lambda i: (i, 0)),
            pl.BlockSpec((K, D), lambda i: (0, 0)),
            pl.BlockSpec((N, D), lambda i: (0, 0)),
        ],
        out_specs=[
            pl.BlockSpec((block_rows, K), lambda i: (i, 0)),
            pl.BlockSpec((block_rows,), lambda i: (i,)),
        ],
        out_shape=[
            jax.ShapeDtypeStruct((B, K), jnp.float32),
            jax.ShapeDtypeStruct((B,), jnp.float32),
        ],
    )(v, vhat, F, negatives)


_CHUNK_OFFS = (0, 16, 32, 48, 64, 80, 84)


def _bottom16_row(g_v, d_v, row, lane):
    chunks = []
    for cki, off in enumerate(_CHUNK_OFFS):
        k = g_v[row, pl.ds(off, 16)]
        v = d_v[row, pl.ds(off, 16)]
        if off % 16:
            k = jnp.where(lane < (16 - K % 16), _INF, k)
        chunks.append(plsc.sort_key_val(k, v))

    def merge(a, b, resort=True):
        ak, av = a
        bk, bv = b
        rk = lax.rev(bk, (0,))
        rv = lax.rev(bv, (0,))
        take_a = ak <= rk
        mk = jnp.where(take_a, ak, rk)
        mv = jnp.where(take_a, av, rv)
        if resort:
            return plsc.sort_key_val(mk, mv)
        return mk, mv

    while len(chunks) > 1:
        last_level = len(chunks) == 2
        nxt = [merge(chunks[i], chunks[i + 1], resort=not last_level)
               for i in range(0, len(chunks) - 1, 2)]
        if len(chunks) % 2:
            nxt.append(chunks[-1])
        chunks = nxt
    return chunks[0]


def _make_sc_stage(B):
    info = plsc.get_sparse_core_info()
    NC, NS = info.num_cores, info.num_subcores
    NW = NC * NS
    RW = B // NW
    CR = 128
    NCH = RW // CR
    GROUPS = CR // 16

    mesh = plsc.VectorSubcoreMesh(core_axis_name="c", subcore_axis_name="s")

    @functools.partial(
        pl.kernel,
        out_type=jax.ShapeDtypeStruct((B,), jnp.float32),
        mesh=mesh,
        compiler_params=pltpu.CompilerParams(needs_layout_passes=False),
        scratch_types=[
            pltpu.VMEM((2, CR, K), jnp.float32),
            pltpu.VMEM((2, CR, K), jnp.float32),
            pltpu.VMEM((RW,), jnp.float32),
            pltpu.VMEM((RW,), jnp.float32),
            pltpu.SemaphoreType.DMA,
            pltpu.SemaphoreType.DMA,
        ],
    )
    def sc_topk(g_hbm, dist_hbm, rest_hbm, out_hbm,
                g_v, d_v, r_v, o_v, sem0, sem1):
        wid = lax.axis_index("s") * NC + lax.axis_index("c")
        row0 = wid * RW
        sems = (sem0, sem1)

        def start(ci, slot):
            rows = pl.ds(row0 + ci * CR, CR)
            dg = pltpu.async_copy(g_hbm.at[rows], g_v.at[slot], sems[slot])
            dd = pltpu.async_copy(dist_hbm.at[rows], d_v.at[slot], sems[slot])
            return dg, dd

        pltpu.sync_copy(rest_hbm.at[pl.ds(row0, RW)], r_v)

        lane = lax.iota(jnp.int32, 16)
        pending = start(0, 0)

        for ci in range(NCH):
            slot = ci % 2
            if ci + 1 < NCH:
                nxt = start(ci + 1, 1 - slot)
            for dsc in pending:
                dsc.wait()
            if ci + 1 < NCH:
                pending = nxt

            def group_body(gi, carry, ci=ci, slot=slot):
                acc = jnp.zeros((16,), jnp.float32)
                for j in range(16):
                    row = gi * 16 + j
                    bk, bv = _bottom16_row(g_v.at[slot], d_v.at[slot],
                                           row, lane)
                    s = jnp.sum(bk)
                    g_t = bk / (s + 1e-10)
                    one_m = 1.0 - g_t
                    m_t = M * one_m * one_m
                    arow = jnp.full((16,), ci * CR + row, jnp.int32)
                    restv = plsc.load_gather(r_v, [arow])
                    hinge = jnp.maximum(m_t - bv, 0.0)
                    jt = jnp.sum(hinge) * (1.0 / T)
                    acc = jnp.where(lane == j, jt + restv, acc)
                o_v[pl.ds(ci * CR + gi * 16, 16)] = acc
                return carry

            lax.fori_loop(0, GROUPS, group_body, 0)
        pltpu.sync_copy(o_v, out_hbm.at[pl.ds(row0, RW)])

    return sc_topk


@functools.partial(jax.jit, static_argnames=("block_rows",))
def _run(v, vhat, g, F, negatives, block_rows=1024):
    B = v.shape[0]
    dist, rest = _dense_stage(v, vhat, F, negatives, block_rows)
    return _make_sc_stage(B)(g, dist, rest)


def kernel(v, vhat, d, g, F, negatives):
    del d
    return _run(v, vhat, g, F, negatives)


# speedup vs baseline: 1.2917x; 1.0127x over previous
import functools

import jax
import jax.numpy as jnp
from jax import lax
from jax.experimental import pallas as pl
from jax.experimental.pallas import tpu as pltpu
from jax.experimental.pallas import tpu_sc as plsc

LAMBDA_ORTHO = 0.0001
M = 1.0
T = 16
K = 100
_INF = float("inf")


def _dense_block(v_ref, vh_ref, f_ref, neg_ref,
                 dist_ref, rest_ref):
    v = v_ref[...]
    vh = vh_ref[...]
    F = f_ref[...]
    neg = neg_ref[...]
    N = neg.shape[0]

    base = jnp.sqrt(jnp.sum((vh - v) ** 2, axis=1, keepdims=True))
    vh_sq = jnp.sum(vh * vh, axis=1, keepdims=True)

    neg_sq = jnp.sum(neg * neg, axis=1)[None, :]
    dot_n = lax.dot_general(vh, neg, (((1,), (1,)), ((), ())),
                            preferred_element_type=jnp.float32)
    neg_dist = jnp.sqrt(jnp.maximum(vh_sq + neg_sq - 2.0 * dot_n, 0.0))
    ju = jnp.sum(jnp.maximum(1.0 + base - neg_dist, 0.0), axis=1) / N

    f_sq = jnp.sum(F * F, axis=1)[None, :]
    dot_f = lax.dot_general(vh, F, (((1,), (1,)), ((), ())),
                            preferred_element_type=jnp.float32)
    dist_f = jnp.sqrt(jnp.maximum(vh_sq + f_sq - 2.0 * dot_f, 0.0))

    gram = lax.dot_general(F, F, (((1,), (1,)), ((), ())),
                           preferred_element_type=jnp.float32)
    r = lax.broadcasted_iota(jnp.int32, (K, K), 0)
    c = lax.broadcasted_iota(jnp.int32, (K, K), 1)
    eye = jnp.where(r == c, 1.0, 0.0).astype(jnp.float32)
    ortho = jnp.sum(jnp.abs(gram - eye))

    dist_ref[...] = dist_f - base
    rest_ref[...] = ju + LAMBDA_ORTHO * ortho * ortho


def _dense_stage(v, vhat, F, negatives, block_rows):
    B, D = v.shape
    N = negatives.shape[0]
    grid = (B // block_rows,)
    return pl.pallas_call(
        _dense_block,
        grid=grid,
        in_specs=[
            pl.BlockSpec((block_rows, D), lambda i: (i, 0)),
            pl.BlockSpec((block_rows, D), lambda i: (i, 0)),
            pl.BlockSpec((K, D), lambda i: (0, 0)),
            pl.BlockSpec((N, D), lambda i: (0, 0)),
        ],
        out_specs=[
            pl.BlockSpec((block_rows, K), lambda i: (i, 0)),
            pl.BlockSpec((block_rows,), lambda i: (i,)),
        ],
        out_shape=[
            jax.ShapeDtypeStruct((B, K), jnp.float32),
            jax.ShapeDtypeStruct((B,), jnp.float32),
        ],
    )(v, vhat, F, negatives)


_CHUNK_OFFS = (0, 16, 32, 48, 64, 80, 84)


def _bottom16_row(g_v, d_v, row, lane):
    chunks = []
    for cki, off in enumerate(_CHUNK_OFFS):
        k = g_v[row, pl.ds(off, 16)]
        v = d_v[row, pl.ds(off, 16)]
        if off % 16:
            k = jnp.where(lane < (16 - K % 16), _INF, k)
        chunks.append(plsc.sort_key_val(k, v))

    def merge(a, b, resort=True):
        ak, av = a
        bk, bv = b
        rk = lax.rev(bk, (0,))
        rv = lax.rev(bv, (0,))
        take_a = ak <= rk
        mk = jnp.where(take_a, ak, rk)
        mv = jnp.where(take_a, av, rv)
        if resort:
            return plsc.sort_key_val(mk, mv)
        return mk, mv

    while len(chunks) > 1:
        last_level = len(chunks) == 2
        nxt = [merge(chunks[i], chunks[i + 1], resort=not last_level)
               for i in range(0, len(chunks) - 1, 2)]
        if len(chunks) % 2:
            nxt.append(chunks[-1])
        chunks = nxt
    return chunks[0]


def _make_sc_stage(B):
    info = plsc.get_sparse_core_info()
    NC, NS = info.num_cores, info.num_subcores
    NW = NC * NS
    RW = B // NW
    CR = 128
    NCH = RW // CR
    GROUPS = CR // 16

    mesh = plsc.VectorSubcoreMesh(core_axis_name="c", subcore_axis_name="s")

    @functools.partial(
        pl.kernel,
        out_type=jax.ShapeDtypeStruct((B,), jnp.float32),
        mesh=mesh,
        compiler_params=pltpu.CompilerParams(needs_layout_passes=False),
        scratch_types=[
            pltpu.VMEM((2, CR, K), jnp.float32),
            pltpu.VMEM((2, CR, K), jnp.float32),
            pltpu.VMEM((RW,), jnp.float32),
            pltpu.VMEM((RW,), jnp.float32),
            pltpu.SemaphoreType.DMA,
            pltpu.SemaphoreType.DMA,
        ],
    )
    def sc_topk(g_hbm, dist_hbm, rest_hbm, out_hbm,
                g_v, d_v, r_v, o_v, sem0, sem1):
        wid = lax.axis_index("s") * NC + lax.axis_index("c")
        row0 = wid * RW
        sems = (sem0, sem1)

        def start(ci, slot):
            rows = pl.ds(row0 + ci * CR, CR)
            dg = pltpu.async_copy(g_hbm.at[rows], g_v.at[slot], sems[slot])
            dd = pltpu.async_copy(dist_hbm.at[rows], d_v.at[slot], sems[slot])
            return dg, dd

        pltpu.sync_copy(rest_hbm.at[pl.ds(row0, RW)], r_v)

        lane = lax.iota(jnp.int32, 16)
        pending = start(0, 0)

        for ci in range(NCH):
            slot = ci % 2
            if ci + 1 < NCH:
                nxt = start(ci + 1, 1 - slot)
            for dsc in pending:
                dsc.wait()
            if ci + 1 < NCH:
                pending = nxt

            def group_body(gi, carry, ci=ci, slot=slot):
                acc = jnp.zeros((16,), jnp.float32)
                for j in range(16):
                    row = gi * 16 + j
                    bk, bv = _bottom16_row(g_v.at[slot], d_v.at[slot],
                                           row, lane)
                    s = jnp.sum(bk)
                    g_t = bk / (s + 1e-10)
                    one_m = 1.0 - g_t
                    m_t = M * one_m * one_m
                    arow = jnp.full((16,), ci * CR + row, jnp.int32)
                    restv = plsc.load_gather(r_v, [arow])
                    hinge = jnp.maximum(m_t - bv, 0.0)
                    jt = jnp.sum(hinge) * (1.0 / T)
                    acc = jnp.where(lane == j, jt + restv, acc)
                o_v[pl.ds(ci * CR + gi * 16, 16)] = acc
                return carry

            lax.fori_loop(0, GROUPS, group_body, 0)
        pltpu.sync_copy(o_v, out_hbm.at[pl.ds(row0, RW)])

    return sc_topk


@functools.partial(jax.jit, static_argnames=("block_rows",))
def _run(v, vhat, g, F, negatives, block_rows=2048):
    B = v.shape[0]
    dist, rest = _dense_stage(v, vhat, F, negatives, block_rows)
    return _make_sc_stage(B)(g, dist, rest)


def kernel(v, vhat, d, g, F, negatives):
    del d
    return _run(v, vhat, g, F, negatives)


# speedup vs baseline: 1.3047x; 1.0100x over previous
import functools

import jax
import jax.numpy as jnp
from jax import lax
from jax.experimental import pallas as pl
from jax.experimental.pallas import tpu as pltpu
from jax.experimental.pallas import tpu_sc as plsc

LAMBDA_ORTHO = 0.0001
M = 1.0
T = 16
K = 100
_INF = float("inf")


def _dense_block(v_ref, vh_ref, f_ref, neg_ref,
                 dist_ref, rest_ref):
    v = v_ref[...]
    vh = vh_ref[...]
    F = f_ref[...]
    neg = neg_ref[...]
    N = neg.shape[0]

    base = jnp.sqrt(jnp.sum((vh - v) ** 2, axis=1, keepdims=True))
    vh_sq = jnp.sum(vh * vh, axis=1, keepdims=True)

    neg_sq = jnp.sum(neg * neg, axis=1)[None, :]
    dot_n = lax.dot_general(vh, neg, (((1,), (1,)), ((), ())),
                            preferred_element_type=jnp.float32)
    neg_dist = jnp.sqrt(jnp.maximum(vh_sq + neg_sq - 2.0 * dot_n, 0.0))
    ju = jnp.sum(jnp.maximum(1.0 + base - neg_dist, 0.0), axis=1) / N

    f_sq = jnp.sum(F * F, axis=1)[None, :]
    dot_f = lax.dot_general(vh, F, (((1,), (1,)), ((), ())),
                            preferred_element_type=jnp.float32)
    dist_f = jnp.sqrt(jnp.maximum(vh_sq + f_sq - 2.0 * dot_f, 0.0))

    gram = lax.dot_general(F, F, (((1,), (1,)), ((), ())),
                           preferred_element_type=jnp.float32)
    r = lax.broadcasted_iota(jnp.int32, (K, K), 0)
    c = lax.broadcasted_iota(jnp.int32, (K, K), 1)
    eye = jnp.where(r == c, 1.0, 0.0).astype(jnp.float32)
    ortho = jnp.sum(jnp.abs(gram - eye))

    dist_ref[...] = dist_f - base
    rest_ref[...] = ju + LAMBDA_ORTHO * ortho * ortho


def _dense_stage(v, vhat, F, negatives, block_rows):
    B, D = v.shape
    N = negatives.shape[0]
    grid = (B // block_rows,)
    return pl.pallas_call(
        _dense_block,
        grid=grid,
        in_specs=[
            pl.BlockSpec((block_rows, D), lambda i: (i, 0)),
            pl.BlockSpec((block_rows, D), lambda i: (i, 0)),
            pl.BlockSpec((K, D), lambda i: (0, 0)),
            pl.BlockSpec((N, D), lambda i: (0, 0)),
        ],
        out_specs=[
            pl.BlockSpec((block_rows, K), lambda i: (i, 0)),
            pl.BlockSpec((block_rows,), lambda i: (i,)),
        ],
        out_shape=[
            jax.ShapeDtypeStruct((B, K), jnp.float32),
            jax.ShapeDtypeStruct((B,), jnp.float32),
        ],
    )(v, vhat, F, negatives)


_CHUNK_OFFS = (0, 16, 32, 48, 64, 80, 84)


def _bottom16_row(g_v, d_v, row, lane):
    dirs = (False, True, False, True, False, True, True)
    srt = []
    for off, desc in zip(_CHUNK_OFFS, dirs):
        k = g_v[row, pl.ds(off, 16)]
        v = d_v[row, pl.ds(off, 16)]
        if off % 16:
            k = jnp.where(lane < (16 - K % 16), _INF, k)
        srt.append(plsc.sort_key_val(k, v, descending=desc))

    def merge(a, b, direction):
        ak, av = a
        bk, bv = b
        take_a = ak <= bk
        mk = jnp.minimum(ak, bk)
        mv = jnp.where(take_a, av, bv)
        if direction is None:
            return mk, mv
        return plsc.sort_key_val(mk, mv, descending=direction)

    m0 = merge(srt[0], srt[1], False)
    m1 = merge(srt[2], srt[3], True)
    m2 = merge(srt[4], srt[5], False)
    m3 = merge(m0, m1, False)
    m4 = merge(m2, srt[6], True)
    return merge(m3, m4, None)


def _make_sc_stage(B):
    info = plsc.get_sparse_core_info()
    NC, NS = info.num_cores, info.num_subcores
    NW = NC * NS
    RW = B // NW
    CR = 128
    NCH = RW // CR
    GROUPS = CR // 16

    mesh = plsc.VectorSubcoreMesh(core_axis_name="c", subcore_axis_name="s")

    @functools.partial(
        pl.kernel,
        out_type=jax.ShapeDtypeStruct((B,), jnp.float32),
        mesh=mesh,
        compiler_params=pltpu.CompilerParams(needs_layout_passes=False),
        scratch_types=[
            pltpu.VMEM((2, CR, K), jnp.float32),
            pltpu.VMEM((2, CR, K), jnp.float32),
            pltpu.VMEM((RW,), jnp.float32),
            pltpu.VMEM((RW,), jnp.float32),
            pltpu.SemaphoreType.DMA,
            pltpu.SemaphoreType.DMA,
        ],
    )
    def sc_topk(g_hbm, dist_hbm, rest_hbm, out_hbm,
                g_v, d_v, r_v, o_v, sem0, sem1):
        wid = lax.axis_index("s") * NC + lax.axis_index("c")
        row0 = wid * RW
        sems = (sem0, sem1)

        def start(ci, slot):
            rows = pl.ds(row0 + ci * CR, CR)
            dg = pltpu.async_copy(g_hbm.at[rows], g_v.at[slot], sems[slot])
            dd = pltpu.async_copy(dist_hbm.at[rows], d_v.at[slot], sems[slot])
            return dg, dd

        pltpu.sync_copy(rest_hbm.at[pl.ds(row0, RW)], r_v)

        lane = lax.iota(jnp.int32, 16)
        pending = start(0, 0)

        for ci in range(NCH):
            slot = ci % 2
            if ci + 1 < NCH:
                nxt = start(ci + 1, 1 - slot)
            for dsc in pending:
                dsc.wait()
            if ci + 1 < NCH:
                pending = nxt

            def group_body(gi, carry, ci=ci, slot=slot):
                acc = jnp.zeros((16,), jnp.float32)
                for j in range(16):
                    row = gi * 16 + j
                    bk, bv = _bottom16_row(g_v.at[slot], d_v.at[slot],
                                           row, lane)
                    s = jnp.sum(bk)
                    g_t = bk / (s + 1e-10)
                    one_m = 1.0 - g_t
                    m_t = M * one_m * one_m
                    hinge = jnp.maximum(m_t - bv, 0.0)
                    jt = jnp.sum(hinge) * (1.0 / T)
                    acc = jnp.where(lane == j, jt, acc)
                restv = r_v[pl.ds(ci * CR + gi * 16, 16)]
                o_v[pl.ds(ci * CR + gi * 16, 16)] = acc + restv
                return carry

            lax.fori_loop(0, GROUPS, group_body, 0)
        pltpu.sync_copy(o_v, out_hbm.at[pl.ds(row0, RW)])

    return sc_topk


@functools.partial(jax.jit, static_argnames=("block_rows",))
def _run(v, vhat, g, F, negatives, block_rows=2048):
    B = v.shape[0]
    dist, rest = _dense_stage(v, vhat, F, negatives, block_rows)
    return _make_sc_stage(B)(g, dist, rest)


def kernel(v, vhat, d, g, F, negatives):
    del d
    return _run(v, vhat, g, F, negatives)


# speedup vs baseline: 1.3244x; 1.0151x over previous
import functools

import jax
import jax.numpy as jnp
from jax import lax
from jax.experimental import pallas as pl
from jax.experimental.pallas import tpu as pltpu
from jax.experimental.pallas import tpu_sc as plsc

LAMBDA_ORTHO = 0.0001
M = 1.0
T = 16
K = 100
_INF = float("inf")


def _dense_block(v_ref, vh_ref, f_ref, neg_ref,
                 dist_ref, rest_ref, ortho_ref):
    v = v_ref[...]
    vh = vh_ref[...]
    F = f_ref[...]
    neg = neg_ref[...]
    D = v.shape[1]
    N = neg.shape[0]

    @pl.when(pl.program_id(0) == 0)
    def _():
        gram = lax.dot_general(F, F, (((1,), (1,)), ((), ())),
                               preferred_element_type=jnp.float32)
        r = lax.broadcasted_iota(jnp.int32, (K, K), 0)
        c = lax.broadcasted_iota(jnp.int32, (K, K), 1)
        eye = jnp.where(r == c, 1.0, 0.0).astype(jnp.float32)
        ortho = jnp.sum(jnp.abs(gram - eye))
        ortho_ref[0, 0] = LAMBDA_ORTHO * ortho * ortho

    ones = jnp.ones((D, 1), jnp.float32)
    diff = vh - v
    base2 = lax.dot_general(diff * diff, ones, (((1,), (0,)), ((), ())),
                            preferred_element_type=jnp.float32)
    base = jnp.sqrt(base2)
    vh_sq = lax.dot_general(vh * vh, ones, (((1,), (0,)), ((), ())),
                            preferred_element_type=jnp.float32)

    neg_sq = jnp.sum(neg * neg, axis=1)[None, :]
    dot_n = lax.dot_general(vh, neg, (((1,), (1,)), ((), ())),
                            preferred_element_type=jnp.float32)
    neg_dist = jnp.sqrt(jnp.maximum(vh_sq + neg_sq - 2.0 * dot_n, 0.0))
    ju = jnp.sum(jnp.maximum(1.0 + base - neg_dist, 0.0), axis=1) / N

    f_sq = jnp.sum(F * F, axis=1)[None, :]
    dot_f = lax.dot_general(vh, F, (((1,), (1,)), ((), ())),
                            preferred_element_type=jnp.float32)
    dist_f = jnp.sqrt(jnp.maximum(vh_sq + f_sq - 2.0 * dot_f, 0.0))

    dist_ref[...] = dist_f - base
    rest_ref[...] = ju + ortho_ref[0, 0]


def _dense_stage(v, vhat, F, negatives, block_rows):
    B, D = v.shape
    N = negatives.shape[0]
    grid = (B // block_rows,)
    return pl.pallas_call(
        _dense_block,
        grid=grid,
        in_specs=[
            pl.BlockSpec((block_rows, D), lambda i: (i, 0)),
            pl.BlockSpec((block_rows, D), lambda i: (i, 0)),
            pl.BlockSpec((K, D), lambda i: (0, 0)),
            pl.BlockSpec((N, D), lambda i: (0, 0)),
        ],
        out_specs=[
            pl.BlockSpec((block_rows, K), lambda i: (i, 0)),
            pl.BlockSpec((block_rows,), lambda i: (i,)),
        ],
        out_shape=[
            jax.ShapeDtypeStruct((B, K), jnp.float32),
            jax.ShapeDtypeStruct((B,), jnp.float32),
        ],
        scratch_shapes=[pltpu.SMEM((1, 1), jnp.float32)],
    )(v, vhat, F, negatives)


_CHUNK_OFFS = (0, 16, 32, 48, 64, 80, 84)


def _bottom16_row(g_v, d_v, row, lane):
    dirs = (False, True, False, True, False, True, True)
    srt = []
    for off, desc in zip(_CHUNK_OFFS, dirs):
        k = g_v[row, pl.ds(off, 16)]
        v = d_v[row, pl.ds(off, 16)]
        if off % 16:
            k = jnp.where(lane < (16 - K % 16), _INF, k)
        srt.append(plsc.sort_key_val(k, v, descending=desc))

    def merge(a, b, direction):
        ak, av = a
        bk, bv = b
        take_a = ak <= bk
        mk = jnp.minimum(ak, bk)
        mv = jnp.where(take_a, av, bv)
        if direction is None:
            return mk, mv
        return plsc.sort_key_val(mk, mv, descending=direction)

    m0 = merge(srt[0], srt[1], False)
    m1 = merge(srt[2], srt[3], True)
    m2 = merge(srt[4], srt[5], False)
    m3 = merge(m0, m1, False)
    m4 = merge(m2, srt[6], True)
    return merge(m3, m4, None)


def _make_sc_stage(B):
    info = plsc.get_sparse_core_info()
    NC, NS = info.num_cores, info.num_subcores
    NW = NC * NS
    RW = B // NW
    CR = 128
    NCH = RW // CR
    GROUPS = CR // 16

    mesh = plsc.VectorSubcoreMesh(core_axis_name="c", subcore_axis_name="s")

    @functools.partial(
        pl.kernel,
        out_type=jax.ShapeDtypeStruct((B,), jnp.float32),
        mesh=mesh,
        compiler_params=pltpu.CompilerParams(needs_layout_passes=False),
        scratch_types=[
            pltpu.VMEM((2, CR, K), jnp.float32),
            pltpu.VMEM((2, CR, K), jnp.float32),
            pltpu.VMEM((RW,), jnp.float32),
            pltpu.VMEM((RW,), jnp.float32),
            pltpu.SemaphoreType.DMA,
            pltpu.SemaphoreType.DMA,
        ],
    )
    def sc_topk(g_hbm, dist_hbm, rest_hbm, out_hbm,
                g_v, d_v, r_v, o_v, sem0, sem1):
        wid = lax.axis_index("s") * NC + lax.axis_index("c")
        row0 = wid * RW
        sems = (sem0, sem1)

        def start(ci, slot):
            rows = pl.ds(row0 + ci * CR, CR)
            dg = pltpu.async_copy(g_hbm.at[rows], g_v.at[slot], sems[slot])
            dd = pltpu.async_copy(dist_hbm.at[rows], d_v.at[slot], sems[slot])
            return dg, dd

        pltpu.sync_copy(rest_hbm.at[pl.ds(row0, RW)], r_v)

        lane = lax.iota(jnp.int32, 16)
        pending = start(0, 0)

        for ci in range(NCH):
            slot = ci % 2
            if ci + 1 < NCH:
                nxt = start(ci + 1, 1 - slot)
            for dsc in pending:
                dsc.wait()
            if ci + 1 < NCH:
                pending = nxt

            def group_body(gi, carry, ci=ci, slot=slot):
                acc = jnp.zeros((16,), jnp.float32)
                for j in range(16):
                    row = gi * 16 + j
                    bk, bv = _bottom16_row(g_v.at[slot], d_v.at[slot],
                                           row, lane)
                    s = jnp.sum(bk)
                    g_t = bk / (s + 1e-10)
                    one_m = 1.0 - g_t
                    m_t = M * one_m * one_m
                    hinge = jnp.maximum(m_t - bv, 0.0)
                    jt = jnp.sum(hinge) * (1.0 / T)
                    acc = jnp.where(lane == j, jt, acc)
                restv = r_v[pl.ds(ci * CR + gi * 16, 16)]
                o_v[pl.ds(ci * CR + gi * 16, 16)] = acc + restv
                return carry

            lax.fori_loop(0, GROUPS, group_body, 0)
        pltpu.sync_copy(o_v, out_hbm.at[pl.ds(row0, RW)])

    return sc_topk


@functools.partial(jax.jit, static_argnames=("block_rows",))
def _run(v, vhat, g, F, negatives, block_rows=2048):
    B = v.shape[0]
    dist, rest = _dense_stage(v, vhat, F, negatives, block_rows)
    return _make_sc_stage(B)(g, dist, rest)


def kernel(v, vhat, d, g, F, negatives):
    del d
    return _run(v, vhat, g, F, negatives)
